# baseline (speedup 1.0000x reference)
import functools

import numpy as np
import jax
import jax.numpy as jnp
from jax.experimental import pallas as pl

R, S, P, K = 512, 16, 2048, 20
RADIUS = 4.0 * 0.05
FEAT = 256
N = R * S

_R2_F32 = np.float32(RADIUS) * np.float32(RADIUS)
_TMAX = int(np.asarray(_R2_F32, np.float32).view(np.int32)) - 1
_BITS = 30

BR = 256


def _emb_cols(offset, dim, nfreq):
    base = list(range(offset, offset + dim))
    sins, coss = [], []
    for i in range(nfreq):
        sins += list(range(offset + dim * (1 + 2 * i), offset + dim * (2 + 2 * i)))
        coss += list(range(offset + dim * (2 + 2 * i), offset + dim * (3 + 2 * i)))
    return base, sins, coss


def _w0_permutation():
    hp = _emb_cols(0, 3, 10)
    den = _emb_cols(63, 1, 4)
    pos = _emb_cols(72, 3, 10)
    var = _emb_cols(135, 3, 10)
    hd = _emb_cols(198, 3, 4)
    sd = _emb_cols(225, 3, 4)
    normals = list(range(252, 255))
    fv = list(range(255, 511))
    perm = []
    for b, s, c in (hp, den, pos, var, hd, sd):
        perm += b + s + c
    perm += normals + fv
    assert len(perm) == 511 and sorted(perm) == list(range(511))
    return np.asarray(perm, np.int32)


_PERM = _w0_permutation()


def _embed_pieces(x, nfreq):
    scaled = jnp.concatenate([x * (2.0 ** i) for i in range(nfreq)], axis=1)
    return [x, jnp.sin(scaled), jnp.cos(scaled)]


def _fused_kernel(pts_ref, ptr_ref, pcl_ref, hdir_ref, ro_ref, nrm_ref,
                  fv_ref, w0_ref, b0_ref, w1_ref, b1_ref, w2_ref, b2_ref,
                  w3_ref, b3_ref, w4_ref, b4_ref, out_ref):
    f32 = jnp.float32
    X = pts_ref[...]
    ptr = ptr_ref[...]
    xn = X[:, 0:1] * X[:, 0:1] + X[:, 1:2] * X[:, 1:2] + X[:, 2:3] * X[:, 2:3]
    pn = jnp.sum(ptr * ptr, axis=0, keepdims=True)
    ip_sel = jax.lax.dot_general(
        X.astype(jnp.bfloat16), ptr.astype(jnp.bfloat16),
        (((1,), (0,)), ((), ())), preferred_element_type=f32)
    d2s = jnp.maximum(xn + pn - 2.0 * ip_sel, 0.0)
    D = jax.lax.bitcast_convert_type(d2s, jnp.int32)
    ip = (X[:, 0:1] * ptr[0:1, :] + X[:, 1:2] * ptr[1:2, :]
          + X[:, 2:3] * ptr[2:3, :])
    d2 = jnp.maximum(xn + pn - 2.0 * ip, 0.0)

    ones_col = jnp.ones((P, 1), f32)
    dot = functools.partial(jax.lax.dot_general,
                            dimension_numbers=(((1,), (0,)), ((), ())),
                            preferred_element_type=f32,
                            precision=jax.lax.Precision.HIGHEST)
    Dq = (jnp.minimum(jax.lax.shift_right_logical(D, 14), 65535)
          - 32768).astype(jnp.int16)

    ones_col16 = jnp.ones((P, 1), jnp.bfloat16)
    one16 = jnp.ones((), jnp.bfloat16)
    zero16 = jnp.zeros((), jnp.bfloat16)

    def cbody(_, carry):
        lo, hi = carry
        mid = jax.lax.shift_right_logical(lo + hi, 1)
        mid16 = (mid - 32768).astype(jnp.int16)
        mask16 = jnp.where(Dq <= mid16, one16, zero16)
        cnt = jax.lax.dot_general(mask16, ones_col16, (((1,), (0,)), ((), ())),
                                  preferred_element_type=f32)
        ge = cnt >= float(K)
        return jnp.where(ge, lo, mid + 1), jnp.where(ge, mid, hi)

    qmax = _TMAX >> 14
    lo0 = jnp.zeros((BR, 1), jnp.int32)
    hi0 = jnp.full((BR, 1), qmax, jnp.int32)
    mq, _ = jax.lax.fori_loop(0, 16, cbody, (lo0, hi0))
    mq = jnp.minimum(mq, qmax)

    def body(_, carry):
        lo, hi = carry
        mid = jax.lax.shift_right_logical(lo + hi, 1)
        cnt = dot(jnp.where(D <= mid, 1.0, 0.0).astype(f32), ones_col)
        ge = cnt >= float(K)
        return jnp.where(ge, lo, mid + 1), jnp.where(ge, mid, hi)

    elo0 = jax.lax.shift_left(mq, 14)
    ehi0 = jnp.minimum(elo0 + 0x3FFF, _TMAX)
    t, _ = jax.lax.fori_loop(0, 14, body, (elo0, ehi0))
    t = jnp.minimum(t, ehi0)

    sel = D <= t
    d = jnp.sqrt(d2 + 1e-12)
    q = d * (1.0 / RADIUS)
    w = jnp.maximum(1.0 - q * q * q, 0.0)
    A = jnp.where(sel, w, 0.0).astype(f32)
    B = jnp.where(sel & (D > 0), 1.0, 0.0).astype(f32)
    csel = jnp.sum(sel.astype(f32), axis=1, keepdims=True)

    pcl = pcl_ref[...]
    M = jnp.concatenate([pcl, pcl * pcl, jnp.ones((P, 1), f32)], axis=1)
    dot = functools.partial(jax.lax.dot_general,
                            dimension_numbers=(((1,), (0,)), ((), ())),
                            preferred_element_type=f32,
                            precision=jax.lax.Precision.HIGHEST)
    GA = dot(A, M)
    GB = dot(B, M)

    npad = jnp.maximum(float(K) - csel, 0.0)
    d0 = jnp.sqrt(xn + 1e-12)
    q0 = d0 * (1.0 / RADIUS)
    w0pad = jnp.maximum(1.0 - q0 * q0 * q0, 0.0)
    density = GA[:, 6:7] + npad * w0pad
    pos = GA[:, 0:3] / (density + 1e-12)
    num_nn = GB[:, 6:7]
    svec = GB[:, 0:3] - num_nn * X
    vmean = svec / (num_nn + 1e-12)
    svec2 = GB[:, 3:6] - 2.0 * X * GB[:, 0:3] + num_nn * (X * X)
    var = (svec2 - 2.0 * vmean * svec + num_nn * vmean * vmean) / (num_nn + 1e-12)

    ro = ro_ref[...]
    sd_raw = pos - ro
    sd = sd_raw / jnp.sqrt(jnp.sum(sd_raw * sd_raw, axis=1, keepdims=True))

    pieces = []
    pieces += _embed_pieces(X, 10)
    pieces += _embed_pieces(density, 4)
    pieces += _embed_pieces(pos, 10)
    pieces += _embed_pieces(var, 10)
    pieces += _embed_pieces(hdir_ref[...], 4)
    pieces += _embed_pieces(sd, 4)
    pieces += [nrm_ref[...], fv_ref[...], jnp.zeros((BR, 1), f32)]
    feat = jnp.concatenate(pieces, axis=1)

    dotd = functools.partial(jax.lax.dot_general,
                             dimension_numbers=(((1,), (0,)), ((), ())),
                             preferred_element_type=f32)
    h = jnp.maximum(dotd(feat, w0_ref[...]) + b0_ref[...], 0.0)
    h = jnp.maximum(dotd(h, w1_ref[...]) + b1_ref[...], 0.0)
    h = jnp.maximum(dotd(h, w2_ref[...]) + b2_ref[...], 0.0)
    h = jnp.maximum(dotd(h, w3_ref[...]) + b3_ref[...], 0.0)
    o = dotd(h, w4_ref[...]) + b4_ref[...]
    out_ref[...] = jax.nn.sigmoid(o)


def kernel(points, normals, view_dirs, feature_vectors, physical_particles,
           rays, ro, W0, b0, W1, b1, W2, b2, W3, b3, W4, b4):
    f32 = jnp.float32
    pts = points.reshape(N, 3)
    ptr = physical_particles.T
    hdir = jnp.repeat(rays[:, 3:], S, axis=0)

    W0t = jnp.concatenate([W0[:, _PERM].T, jnp.zeros((1, 512), f32)], axis=0)
    W4t = jnp.concatenate([W4.T, jnp.zeros((512, 125), f32)], axis=1)
    b4p = jnp.concatenate([b4, jnp.zeros((125,), f32)])

    grid = (N // BR,)
    row = lambda i: (i, 0)
    rep = lambda i: (0, 0)
    out = pl.pallas_call(
        _fused_kernel,
        grid=grid,
        in_specs=[
            pl.BlockSpec((BR, 3), row),
            pl.BlockSpec((3, P), rep),
            pl.BlockSpec((P, 3), rep),
            pl.BlockSpec((BR, 3), row),
            pl.BlockSpec((1, 3), rep),
            pl.BlockSpec((BR, 3), row),
            pl.BlockSpec((BR, FEAT), row),
            pl.BlockSpec((512, 512), rep),
            pl.BlockSpec((1, 512), rep),
            pl.BlockSpec((512, 512), rep),
            pl.BlockSpec((1, 512), rep),
            pl.BlockSpec((512, 512), rep),
            pl.BlockSpec((1, 512), rep),
            pl.BlockSpec((512, 512), rep),
            pl.BlockSpec((1, 512), rep),
            pl.BlockSpec((512, 128), rep),
            pl.BlockSpec((1, 128), rep),
        ],
        out_specs=pl.BlockSpec((BR, 128), row),
        out_shape=jax.ShapeDtypeStruct((N, 128), f32),
    )(pts, ptr, physical_particles, hdir, ro, normals, feature_vectors,
      W0t, b0.reshape(1, 512), W1.T, b1.reshape(1, 512), W2.T,
      b2.reshape(1, 512), W3.T, b3.reshape(1, 512), W4t, b4p.reshape(1, 128))
    return out[:, :3]


# speedup vs baseline: 2.5337x; 2.5337x over previous
import functools

import numpy as np
import jax
import jax.numpy as jnp
from jax.experimental import pallas as pl

R, S, P, K = 512, 16, 2048, 20
RADIUS = 4.0 * 0.05
FEAT = 256
N = R * S

_R2_F32 = np.float32(RADIUS) * np.float32(RADIUS)
_TMAX = int(np.asarray(_R2_F32, np.float32).view(np.int32)) - 1
_BITS = 30

BR = 256


def _emb_cols(offset, dim, nfreq):
    base = list(range(offset, offset + dim))
    sins, coss = [], []
    for i in range(nfreq):
        sins += list(range(offset + dim * (1 + 2 * i), offset + dim * (2 + 2 * i)))
        coss += list(range(offset + dim * (2 + 2 * i), offset + dim * (3 + 2 * i)))
    return base, sins, coss


def _w0_permutation():
    hp = _emb_cols(0, 3, 10)
    den = _emb_cols(63, 1, 4)
    pos = _emb_cols(72, 3, 10)
    var = _emb_cols(135, 3, 10)
    hd = _emb_cols(198, 3, 4)
    sd = _emb_cols(225, 3, 4)
    normals = list(range(252, 255))
    fv = list(range(255, 511))
    perm = []
    for b, s, c in (hp, den, pos, var, hd, sd):
        perm += b + s + c
    perm += normals + fv
    assert len(perm) == 511 and sorted(perm) == list(range(511))
    return np.asarray(perm, np.int32)


_PERM = _w0_permutation()


def _embed_pieces(x, nfreq):
    scaled = jnp.concatenate([x * (2.0 ** i) for i in range(nfreq)], axis=1)
    return [x, jnp.sin(scaled), jnp.cos(scaled)]


def _fused_kernel(pts_ref, ptr_ref, pcl_ref, hdir_ref, ro_ref, nrm_ref,
                  fv_ref, w0_ref, b0_ref, w1_ref, b1_ref, w2_ref, b2_ref,
                  w3_ref, b3_ref, w4_ref, b4_ref, out_ref):
    f32 = jnp.float32
    X = pts_ref[...]
    ptr = ptr_ref[...]
    xn = X[:, 0:1] * X[:, 0:1] + X[:, 1:2] * X[:, 1:2] + X[:, 2:3] * X[:, 2:3]
    pn = jnp.sum(ptr * ptr, axis=0, keepdims=True)
    ip_sel = jax.lax.dot_general(
        X.astype(jnp.bfloat16), ptr.astype(jnp.bfloat16),
        (((1,), (0,)), ((), ())), preferred_element_type=f32)
    d2s = jnp.maximum(xn + pn - 2.0 * ip_sel, 0.0)
    D = jax.lax.bitcast_convert_type(d2s, jnp.int32)
    ip = (X[:, 0:1] * ptr[0:1, :] + X[:, 1:2] * ptr[1:2, :]
          + X[:, 2:3] * ptr[2:3, :])
    d2 = jnp.maximum(xn + pn - 2.0 * ip, 0.0)

    def body(_, carry):
        lo, hi = carry
        mid = jax.lax.shift_right_logical(lo + hi, 1)
        cnt = jnp.sum((D <= mid).astype(f32), axis=1, keepdims=True)
        ge = cnt >= float(K)
        return jnp.where(ge, lo, mid + 1), jnp.where(ge, mid, hi)

    lo0 = jnp.zeros((BR, 1), jnp.int32)
    hi0 = jnp.full((BR, 1), _TMAX, jnp.int32)
    t, _ = jax.lax.fori_loop(0, _BITS, body, (lo0, hi0))
    t = jnp.minimum(t, _TMAX)

    sel = D <= t
    d = jnp.sqrt(d2 + 1e-12)
    q = d * (1.0 / RADIUS)
    w = jnp.maximum(1.0 - q * q * q, 0.0)
    A = jnp.where(sel, w, 0.0).astype(f32)
    B = jnp.where(sel & (D > 0), 1.0, 0.0).astype(f32)
    csel = jnp.sum(sel.astype(f32), axis=1, keepdims=True)

    pcl = pcl_ref[...]
    M = jnp.concatenate([pcl, pcl * pcl, jnp.ones((P, 1), f32)], axis=1)
    dot = functools.partial(jax.lax.dot_general,
                            dimension_numbers=(((1,), (0,)), ((), ())),
                            preferred_element_type=f32)
    GAB = dot(jnp.concatenate([A, B], axis=0), M)
    GA = GAB[:BR]
    GB = GAB[BR:]

    npad = jnp.maximum(float(K) - csel, 0.0)
    d0 = jnp.sqrt(xn + 1e-12)
    q0 = d0 * (1.0 / RADIUS)
    w0pad = jnp.maximum(1.0 - q0 * q0 * q0, 0.0)
    density = GA[:, 6:7] + npad * w0pad
    pos = GA[:, 0:3] / (density + 1e-12)
    num_nn = GB[:, 6:7]
    svec = GB[:, 0:3] - num_nn * X
    vmean = svec / (num_nn + 1e-12)
    svec2 = GB[:, 3:6] - 2.0 * X * GB[:, 0:3] + num_nn * (X * X)
    var = (svec2 - 2.0 * vmean * svec + num_nn * vmean * vmean) / (num_nn + 1e-12)

    ro = ro_ref[...]
    sd_raw = pos - ro
    sd = sd_raw / jnp.sqrt(jnp.sum(sd_raw * sd_raw, axis=1, keepdims=True))

    pieces = []
    pieces += _embed_pieces(X, 10)
    pieces += _embed_pieces(density, 4)
    pieces += _embed_pieces(pos, 10)
    pieces += _embed_pieces(var, 10)
    pieces += _embed_pieces(hdir_ref[...], 4)
    pieces += _embed_pieces(sd, 4)
    pieces += [nrm_ref[...], fv_ref[...], jnp.zeros((BR, 1), f32)]
    feat = jnp.concatenate(pieces, axis=1)

    dotd = functools.partial(jax.lax.dot_general,
                             dimension_numbers=(((1,), (0,)), ((), ())),
                             preferred_element_type=f32)
    h = jnp.maximum(dotd(feat, w0_ref[...]) + b0_ref[...], 0.0)
    h = jnp.maximum(dotd(h, w1_ref[...]) + b1_ref[...], 0.0)
    h = jnp.maximum(dotd(h, w2_ref[...]) + b2_ref[...], 0.0)
    h = jnp.maximum(dotd(h, w3_ref[...]) + b3_ref[...], 0.0)
    o = dotd(h, w4_ref[...]) + b4_ref[...]
    out_ref[...] = jax.nn.sigmoid(o)


def kernel(points, normals, view_dirs, feature_vectors, physical_particles,
           rays, ro, W0, b0, W1, b1, W2, b2, W3, b3, W4, b4):
    f32 = jnp.float32
    pts = points.reshape(N, 3)
    ptr = physical_particles.T
    hdir = jnp.repeat(rays[:, 3:], S, axis=0)

    W0t = jnp.concatenate([W0[:, _PERM].T, jnp.zeros((1, 512), f32)], axis=0)
    W4t = jnp.concatenate([W4.T, jnp.zeros((512, 125), f32)], axis=1)
    b4p = jnp.concatenate([b4, jnp.zeros((125,), f32)])

    grid = (N // BR,)
    row = lambda i: (i, 0)
    rep = lambda i: (0, 0)
    out = pl.pallas_call(
        _fused_kernel,
        grid=grid,
        in_specs=[
            pl.BlockSpec((BR, 3), row),
            pl.BlockSpec((3, P), rep),
            pl.BlockSpec((P, 3), rep),
            pl.BlockSpec((BR, 3), row),
            pl.BlockSpec((1, 3), rep),
            pl.BlockSpec((BR, 3), row),
            pl.BlockSpec((BR, FEAT), row),
            pl.BlockSpec((512, 512), rep),
            pl.BlockSpec((1, 512), rep),
            pl.BlockSpec((512, 512), rep),
            pl.BlockSpec((1, 512), rep),
            pl.BlockSpec((512, 512), rep),
            pl.BlockSpec((1, 512), rep),
            pl.BlockSpec((512, 512), rep),
            pl.BlockSpec((1, 512), rep),
            pl.BlockSpec((512, 128), rep),
            pl.BlockSpec((1, 128), rep),
        ],
        out_specs=pl.BlockSpec((BR, 128), row),
        out_shape=jax.ShapeDtypeStruct((N, 128), f32),
    )(pts, ptr, physical_particles, hdir, ro, normals, feature_vectors,
      W0t, b0.reshape(1, 512), W1.T, b1.reshape(1, 512), W2.T,
      b2.reshape(1, 512), W3.T, b3.reshape(1, 512), W4t, b4p.reshape(1, 128))
    return out[:, :3]


# speedup vs baseline: 2.7507x; 1.0857x over previous
import functools

import numpy as np
import jax
import jax.numpy as jnp
from jax.experimental import pallas as pl

R, S, P, K = 512, 16, 2048, 20
RADIUS = 4.0 * 0.05
FEAT = 256
N = R * S

_R2_F32 = np.float32(RADIUS) * np.float32(RADIUS)
_TMAX = int(np.asarray(_R2_F32, np.float32).view(np.int32)) - 1
_BITS = 30

BR = 256


def _emb_cols(offset, dim, nfreq):
    base = list(range(offset, offset + dim))
    sins, coss = [], []
    for i in range(nfreq):
        sins += list(range(offset + dim * (1 + 2 * i), offset + dim * (2 + 2 * i)))
        coss += list(range(offset + dim * (2 + 2 * i), offset + dim * (3 + 2 * i)))
    return base, sins, coss


_GROUPS = ((0, 3, 10), (63, 1, 4), (72, 3, 10), (135, 3, 10),
           (198, 3, 4), (225, 3, 4))


def _w0_permutation():
    perm = []
    for off, dg, _ in _GROUPS:
        perm += [off + d for d in range(dg)]
    for trig in (1, 2):
        for off, dg, nf in _GROUPS:
            for i in range(nf):
                perm += [off + dg * (trig + 2 * i) + d for d in range(dg)]
    perm += list(range(252, 255)) + list(range(255, 511))
    assert len(perm) == 511 and sorted(perm) == list(range(511))
    return np.asarray(perm, np.int32)


_PERM = _w0_permutation()


def _fused_kernel(pts_ref, ptr_ref, pcl_ref, hdir_ref, ro_ref, nrm_ref,
                  fv_ref, w0_ref, b0_ref, w1_ref, b1_ref, w2_ref, b2_ref,
                  w3_ref, b3_ref, w4_ref, b4_ref, out_ref):
    f32 = jnp.float32
    X = pts_ref[...]
    ptr = ptr_ref[...]
    xn = X[:, 0:1] * X[:, 0:1] + X[:, 1:2] * X[:, 1:2] + X[:, 2:3] * X[:, 2:3]
    pn = jnp.sum(ptr * ptr, axis=0, keepdims=True)
    ip_sel = jax.lax.dot_general(
        X.astype(jnp.bfloat16), ptr.astype(jnp.bfloat16),
        (((1,), (0,)), ((), ())), preferred_element_type=f32)
    d2s = jnp.maximum(xn + pn - 2.0 * ip_sel, 0.0)
    D = jax.lax.bitcast_convert_type(d2s, jnp.int32)
    ip = (X[:, 0:1] * ptr[0:1, :] + X[:, 1:2] * ptr[1:2, :]
          + X[:, 2:3] * ptr[2:3, :])
    d2 = jnp.maximum(xn + pn - 2.0 * ip, 0.0)

    def body(_, carry):
        lo, hi = carry
        mid = jax.lax.shift_right_logical(lo + hi, 1)
        cnt = jnp.sum((D <= mid).astype(f32), axis=1, keepdims=True)
        ge = cnt >= float(K)
        return jnp.where(ge, lo, mid + 1), jnp.where(ge, mid, hi)

    lo0 = jnp.zeros((BR, 1), jnp.int32)
    hi0 = jnp.full((BR, 1), _TMAX, jnp.int32)
    t, _ = jax.lax.fori_loop(0, _BITS, body, (lo0, hi0))
    t = jnp.minimum(t, _TMAX)

    sel = D <= t
    d = jnp.sqrt(d2 + 1e-12)
    q = d * (1.0 / RADIUS)
    w = jnp.maximum(1.0 - q * q * q, 0.0)
    A = jnp.where(sel, w, 0.0).astype(f32)
    B = jnp.where(sel & (D > 0), 1.0, 0.0).astype(f32)
    csel = jnp.sum(sel.astype(f32), axis=1, keepdims=True)

    pcl = pcl_ref[...]
    M = jnp.concatenate([pcl, pcl * pcl, jnp.ones((P, 1), f32)], axis=1)
    dot = functools.partial(jax.lax.dot_general,
                            dimension_numbers=(((1,), (0,)), ((), ())),
                            preferred_element_type=f32)
    GAB = dot(jnp.concatenate([A, B], axis=0), M)
    GA = GAB[:BR]
    GB = GAB[BR:]

    npad = jnp.maximum(float(K) - csel, 0.0)
    d0 = jnp.sqrt(xn + 1e-12)
    q0 = d0 * (1.0 / RADIUS)
    w0pad = jnp.maximum(1.0 - q0 * q0 * q0, 0.0)
    density = GA[:, 6:7] + npad * w0pad
    pos = GA[:, 0:3] / (density + 1e-12)
    num_nn = GB[:, 6:7]
    svec = GB[:, 0:3] - num_nn * X
    vmean = svec / (num_nn + 1e-12)
    svec2 = GB[:, 3:6] - 2.0 * X * GB[:, 0:3] + num_nn * (X * X)
    var = (svec2 - 2.0 * vmean * svec + num_nn * vmean * vmean) / (num_nn + 1e-12)

    ro = ro_ref[...]
    sd_raw = pos - ro
    sd = sd_raw / jnp.sqrt(jnp.sum(sd_raw * sd_raw, axis=1, keepdims=True))

    hdir = hdir_ref[...]
    bases = (X, density, pos, var, hdir, sd)
    scaled = jnp.concatenate(
        [b * (2.0 ** i) for b, (_, _, nf) in zip(bases, _GROUPS)
         for i in range(nf)], axis=1)
    feat = jnp.concatenate(
        list(bases) + [jnp.sin(scaled), jnp.cos(scaled),
                       nrm_ref[...], fv_ref[...], jnp.zeros((BR, 1), f32)],
        axis=1)

    dotd = functools.partial(jax.lax.dot_general,
                             dimension_numbers=(((1,), (0,)), ((), ())),
                             preferred_element_type=f32)
    h = jnp.maximum(dotd(feat, w0_ref[...]) + b0_ref[...], 0.0)
    h = jnp.maximum(dotd(h, w1_ref[...]) + b1_ref[...], 0.0)
    h = jnp.maximum(dotd(h, w2_ref[...]) + b2_ref[...], 0.0)
    h = jnp.maximum(dotd(h, w3_ref[...]) + b3_ref[...], 0.0)
    o = dotd(h, w4_ref[...]) + b4_ref[...]
    out_ref[...] = jax.nn.sigmoid(o)


def kernel(points, normals, view_dirs, feature_vectors, physical_particles,
           rays, ro, W0, b0, W1, b1, W2, b2, W3, b3, W4, b4):
    f32 = jnp.float32
    pts = points.reshape(N, 3)
    ptr = physical_particles.T
    hdir = jnp.repeat(rays[:, 3:], S, axis=0)

    W0t = jnp.concatenate([W0[:, _PERM].T, jnp.zeros((1, 512), f32)], axis=0)
    W4t = jnp.concatenate([W4.T, jnp.zeros((512, 125), f32)], axis=1)
    b4p = jnp.concatenate([b4, jnp.zeros((125,), f32)])

    grid = (N // BR,)
    row = lambda i: (i, 0)
    rep = lambda i: (0, 0)
    out = pl.pallas_call(
        _fused_kernel,
        grid=grid,
        in_specs=[
            pl.BlockSpec((BR, 3), row),
            pl.BlockSpec((3, P), rep),
            pl.BlockSpec((P, 3), rep),
            pl.BlockSpec((BR, 3), row),
            pl.BlockSpec((1, 3), rep),
            pl.BlockSpec((BR, 3), row),
            pl.BlockSpec((BR, FEAT), row),
            pl.BlockSpec((512, 512), rep),
            pl.BlockSpec((1, 512), rep),
            pl.BlockSpec((512, 512), rep),
            pl.BlockSpec((1, 512), rep),
            pl.BlockSpec((512, 512), rep),
            pl.BlockSpec((1, 512), rep),
            pl.BlockSpec((512, 512), rep),
            pl.BlockSpec((1, 512), rep),
            pl.BlockSpec((512, 128), rep),
            pl.BlockSpec((1, 128), rep),
        ],
        out_specs=pl.BlockSpec((BR, 128), row),
        out_shape=jax.ShapeDtypeStruct((N, 128), f32),
    )(pts, ptr, physical_particles, hdir, ro, normals, feature_vectors,
      W0t, b0.reshape(1, 512), W1.T, b1.reshape(1, 512), W2.T,
      b2.reshape(1, 512), W3.T, b3.reshape(1, 512), W4t, b4p.reshape(1, 128))
    return out[:, :3]


# speedup vs baseline: 2.8754x; 1.0453x over previous
import functools

import numpy as np
import jax
import jax.numpy as jnp
from jax.experimental import pallas as pl

R, S, P, K = 512, 16, 2048, 20
RADIUS = 4.0 * 0.05
FEAT = 256
N = R * S

_R2_F32 = np.float32(RADIUS) * np.float32(RADIUS)
_TMAX = int(np.asarray(_R2_F32, np.float32).view(np.int32)) - 1
_BITS = 30

BR = 256


def _emb_cols(offset, dim, nfreq):
    base = list(range(offset, offset + dim))
    sins, coss = [], []
    for i in range(nfreq):
        sins += list(range(offset + dim * (1 + 2 * i), offset + dim * (2 + 2 * i)))
        coss += list(range(offset + dim * (2 + 2 * i), offset + dim * (3 + 2 * i)))
    return base, sins, coss


_GROUPS = ((0, 3, 10), (63, 1, 4), (72, 3, 10), (135, 3, 10),
           (198, 3, 4), (225, 3, 4))


def _w0_permutation():
    perm = []
    for off, dg, _ in _GROUPS:
        perm += [off + d for d in range(dg)]
    for trig in (1, 2):
        for off, dg, nf in _GROUPS:
            for i in range(nf):
                perm += [off + dg * (trig + 2 * i) + d for d in range(dg)]
    perm += list(range(252, 255)) + list(range(255, 511))
    assert len(perm) == 511 and sorted(perm) == list(range(511))
    return np.asarray(perm, np.int32)


_PERM = _w0_permutation()


def _fused_kernel(pts_ref, ptr_ref, pcl_ref, hdir_ref, ro_ref, nrm_ref,
                  fv_ref, w0_ref, b0_ref, w1_ref, b1_ref, w2_ref, b2_ref,
                  w3_ref, b3_ref, w4_ref, b4_ref, out_ref):
    f32 = jnp.float32
    X = pts_ref[...]
    ptr = ptr_ref[...]
    xn = X[:, 0:1] * X[:, 0:1] + X[:, 1:2] * X[:, 1:2] + X[:, 2:3] * X[:, 2:3]
    pn = jnp.sum(ptr * ptr, axis=0, keepdims=True)
    ip_sel = jax.lax.dot_general(
        X.astype(jnp.bfloat16), ptr.astype(jnp.bfloat16),
        (((1,), (0,)), ((), ())), preferred_element_type=f32)
    d2s = jnp.maximum(xn + pn - 2.0 * ip_sel, 0.0)
    D = jax.lax.bitcast_convert_type(d2s, jnp.int32)
    ip = (X[:, 0:1] * ptr[0:1, :] + X[:, 1:2] * ptr[1:2, :]
          + X[:, 2:3] * ptr[2:3, :])
    d2 = jnp.maximum(xn + pn - 2.0 * ip, 0.0)

    one16 = jnp.ones((), jnp.bfloat16)
    zero16 = jnp.zeros((), jnp.bfloat16)
    Dq = (jnp.minimum(jax.lax.shift_right_logical(D, 14), 65535)
          - 32768).astype(jnp.int16)

    def cbody(_, carry):
        lo, hi = carry
        mid = jax.lax.shift_right_logical(lo + hi, 1)
        mid16 = (mid - 32768).astype(jnp.int16)
        mask16 = jnp.where(Dq <= mid16, one16, zero16)
        part = mask16[:, 0:128]
        for j in range(1, 16):
            part = part + mask16[:, 128 * j:128 * (j + 1)]
        cnt = jnp.sum(part.astype(f32), axis=1, keepdims=True)
        ge = cnt >= float(K)
        return jnp.where(ge, lo, mid + 1), jnp.where(ge, mid, hi)

    qmax = _TMAX >> 14
    lo0 = jnp.zeros((BR, 1), jnp.int32)
    hi0 = jnp.full((BR, 1), qmax, jnp.int32)
    mq, _ = jax.lax.fori_loop(0, 16, cbody, (lo0, hi0))
    mq = jnp.minimum(mq, qmax)

    def body(_, carry):
        lo, hi = carry
        mid = jax.lax.shift_right_logical(lo + hi, 1)
        cnt = jnp.sum((D <= mid).astype(f32), axis=1, keepdims=True)
        ge = cnt >= float(K)
        return jnp.where(ge, lo, mid + 1), jnp.where(ge, mid, hi)

    elo0 = jax.lax.shift_left(mq, 14)
    ehi0 = jnp.minimum(elo0 + 0x3FFF, _TMAX)
    t, _ = jax.lax.fori_loop(0, 14, body, (elo0, ehi0))
    t = jnp.minimum(t, ehi0)

    sel = D <= t
    d = jnp.sqrt(d2 + 1e-12)
    q = d * (1.0 / RADIUS)
    w = jnp.maximum(1.0 - q * q * q, 0.0)
    A = jnp.where(sel, w, 0.0).astype(f32)
    B = jnp.where(sel & (D > 0), 1.0, 0.0).astype(f32)
    csel = jnp.sum(sel.astype(f32), axis=1, keepdims=True)

    pcl = pcl_ref[...]
    M = jnp.concatenate([pcl, pcl * pcl, jnp.ones((P, 1), f32)], axis=1)
    dot = functools.partial(jax.lax.dot_general,
                            dimension_numbers=(((1,), (0,)), ((), ())),
                            preferred_element_type=f32)
    GAB = dot(jnp.concatenate([A, B], axis=0), M)
    GA = GAB[:BR]
    GB = GAB[BR:]

    npad = jnp.maximum(float(K) - csel, 0.0)
    d0 = jnp.sqrt(xn + 1e-12)
    q0 = d0 * (1.0 / RADIUS)
    w0pad = jnp.maximum(1.0 - q0 * q0 * q0, 0.0)
    density = GA[:, 6:7] + npad * w0pad
    pos = GA[:, 0:3] / (density + 1e-12)
    num_nn = GB[:, 6:7]
    svec = GB[:, 0:3] - num_nn * X
    vmean = svec / (num_nn + 1e-12)
    svec2 = GB[:, 3:6] - 2.0 * X * GB[:, 0:3] + num_nn * (X * X)
    var = (svec2 - 2.0 * vmean * svec + num_nn * vmean * vmean) / (num_nn + 1e-12)

    ro = ro_ref[...]
    sd_raw = pos - ro
    sd = sd_raw / jnp.sqrt(jnp.sum(sd_raw * sd_raw, axis=1, keepdims=True))

    hdir = hdir_ref[...]
    bases = (X, density, pos, var, hdir, sd)
    scaled = jnp.concatenate(
        [b * (2.0 ** i) for b, (_, _, nf) in zip(bases, _GROUPS)
         for i in range(nf)], axis=1)
    feat = jnp.concatenate(
        list(bases) + [jnp.sin(scaled), jnp.cos(scaled),
                       nrm_ref[...], fv_ref[...], jnp.zeros((BR, 1), f32)],
        axis=1)

    dotd = functools.partial(jax.lax.dot_general,
                             dimension_numbers=(((1,), (0,)), ((), ())),
                             preferred_element_type=f32)
    h = jnp.maximum(dotd(feat, w0_ref[...]) + b0_ref[...], 0.0)
    h = jnp.maximum(dotd(h, w1_ref[...]) + b1_ref[...], 0.0)
    h = jnp.maximum(dotd(h, w2_ref[...]) + b2_ref[...], 0.0)
    h = jnp.maximum(dotd(h, w3_ref[...]) + b3_ref[...], 0.0)
    o = dotd(h, w4_ref[...]) + b4_ref[...]
    out_ref[...] = jax.nn.sigmoid(o)


def kernel(points, normals, view_dirs, feature_vectors, physical_particles,
           rays, ro, W0, b0, W1, b1, W2, b2, W3, b3, W4, b4):
    f32 = jnp.float32
    pts = points.reshape(N, 3)
    ptr = physical_particles.T
    hdir = jnp.repeat(rays[:, 3:], S, axis=0)

    W0t = jnp.concatenate([W0[:, _PERM].T, jnp.zeros((1, 512), f32)], axis=0)
    W4t = jnp.concatenate([W4.T, jnp.zeros((512, 125), f32)], axis=1)
    b4p = jnp.concatenate([b4, jnp.zeros((125,), f32)])

    grid = (N // BR,)
    row = lambda i: (i, 0)
    rep = lambda i: (0, 0)
    out = pl.pallas_call(
        _fused_kernel,
        grid=grid,
        in_specs=[
            pl.BlockSpec((BR, 3), row),
            pl.BlockSpec((3, P), rep),
            pl.BlockSpec((P, 3), rep),
            pl.BlockSpec((BR, 3), row),
            pl.BlockSpec((1, 3), rep),
            pl.BlockSpec((BR, 3), row),
            pl.BlockSpec((BR, FEAT), row),
            pl.BlockSpec((512, 512), rep),
            pl.BlockSpec((1, 512), rep),
            pl.BlockSpec((512, 512), rep),
            pl.BlockSpec((1, 512), rep),
            pl.BlockSpec((512, 512), rep),
            pl.BlockSpec((1, 512), rep),
            pl.BlockSpec((512, 512), rep),
            pl.BlockSpec((1, 512), rep),
            pl.BlockSpec((512, 128), rep),
            pl.BlockSpec((1, 128), rep),
        ],
        out_specs=pl.BlockSpec((BR, 128), row),
        out_shape=jax.ShapeDtypeStruct((N, 128), f32),
    )(pts, ptr, physical_particles, hdir, ro, normals, feature_vectors,
      W0t, b0.reshape(1, 512), W1.T, b1.reshape(1, 512), W2.T,
      b2.reshape(1, 512), W3.T, b3.reshape(1, 512), W4t, b4p.reshape(1, 128))
    return out[:, :3]


# speedup vs baseline: 2.8890x; 1.0047x over previous
import functools

import numpy as np
import jax
import jax.numpy as jnp
from jax.experimental import pallas as pl

R, S, P, K = 512, 16, 2048, 20
RADIUS = 4.0 * 0.05
FEAT = 256
N = R * S

_R2_F32 = np.float32(RADIUS) * np.float32(RADIUS)
_TMAX = int(np.asarray(_R2_F32, np.float32).view(np.int32)) - 1
_BITS = 30

BR = 256


def _emb_cols(offset, dim, nfreq):
    base = list(range(offset, offset + dim))
    sins, coss = [], []
    for i in range(nfreq):
        sins += list(range(offset + dim * (1 + 2 * i), offset + dim * (2 + 2 * i)))
        coss += list(range(offset + dim * (2 + 2 * i), offset + dim * (3 + 2 * i)))
    return base, sins, coss


_GROUPS = ((0, 3, 10), (63, 1, 4), (72, 3, 10), (135, 3, 10),
           (198, 3, 4), (225, 3, 4))


def _w0_permutation():
    perm = []
    for off, dg, _ in _GROUPS:
        perm += [off + d for d in range(dg)]
    for trig in (1, 2):
        for off, dg, nf in _GROUPS:
            for i in range(nf):
                perm += [off + dg * (trig + 2 * i) + d for d in range(dg)]
    perm += list(range(252, 255)) + list(range(255, 511))
    assert len(perm) == 511 and sorted(perm) == list(range(511))
    return np.asarray(perm, np.int32)


_PERM = _w0_permutation()


def _fused_kernel(pts_ref, ptr_ref, pcl_ref, hdir_ref, ro_ref, nrm_ref,
                  fv_ref, w0_ref, b0_ref, w1_ref, b1_ref, w2_ref, b2_ref,
                  w3_ref, b3_ref, w4_ref, b4_ref, out_ref):
    f32 = jnp.float32
    X = pts_ref[...]
    ptr = ptr_ref[...]
    xn = X[:, 0:1] * X[:, 0:1] + X[:, 1:2] * X[:, 1:2] + X[:, 2:3] * X[:, 2:3]
    pn = jnp.sum(ptr * ptr, axis=0, keepdims=True)
    ip_sel = jax.lax.dot_general(
        X.astype(jnp.bfloat16), ptr.astype(jnp.bfloat16),
        (((1,), (0,)), ((), ())), preferred_element_type=f32)
    d2s = jnp.maximum(xn + pn - 2.0 * ip_sel, 0.0)
    D = jax.lax.bitcast_convert_type(d2s, jnp.int32)
    ip = (X[:, 0:1] * ptr[0:1, :] + X[:, 1:2] * ptr[1:2, :]
          + X[:, 2:3] * ptr[2:3, :])
    d2 = jnp.maximum(xn + pn - 2.0 * ip, 0.0)

    one16 = jnp.ones((), jnp.bfloat16)
    zero16 = jnp.zeros((), jnp.bfloat16)
    Dq = (jnp.minimum(jax.lax.shift_right_logical(D, 14), 65535)
          - 32768).astype(jnp.int16)

    def cbody(_, carry):
        lo, hi = carry
        mid = jax.lax.shift_right_logical(lo + hi, 1)
        mid16 = (mid - 32768).astype(jnp.int16)
        mask16 = jnp.where(Dq <= mid16, one16, zero16)
        part = mask16[:, 0:128]
        for j in range(1, 16):
            part = part + mask16[:, 128 * j:128 * (j + 1)]
        cnt = jnp.sum(part.astype(f32), axis=1, keepdims=True)
        ge = cnt >= float(K)
        return jnp.where(ge, lo, mid + 1), jnp.where(ge, mid, hi)

    qmax = _TMAX >> 14
    lo0 = jnp.zeros((BR, 1), jnp.int32)
    hi0 = jnp.full((BR, 1), qmax, jnp.int32)
    mq, _ = jax.lax.fori_loop(0, 16, cbody, (lo0, hi0))
    mq = jnp.minimum(mq, qmax)

    elo0 = jax.lax.shift_left(mq, 14)
    ehi0 = jnp.minimum(elo0 + 0x3FFF, _TMAX)
    El = jnp.where(D < elo0, -32768,
                   jnp.where(D <= ehi0, (D & 0x3FFF) - 16384,
                             32767)).astype(jnp.int16)

    def body(_, carry):
        lo, hi = carry
        mid = jax.lax.shift_right_logical(lo + hi, 1)
        mid16 = (mid - 16384).astype(jnp.int16)
        mask16 = jnp.where(El <= mid16, one16, zero16)
        part = mask16[:, 0:128]
        for j in range(1, 16):
            part = part + mask16[:, 128 * j:128 * (j + 1)]
        cnt = jnp.sum(part.astype(f32), axis=1, keepdims=True)
        ge = cnt >= float(K)
        return jnp.where(ge, lo, mid + 1), jnp.where(ge, mid, hi)

    tl0 = jnp.zeros((BR, 1), jnp.int32)
    th0 = jnp.full((BR, 1), 0x3FFF, jnp.int32)
    tl, _ = jax.lax.fori_loop(0, 14, body, (tl0, th0))
    t = jnp.minimum(elo0 + tl, ehi0)

    sel = D <= t
    d = jnp.sqrt(d2 + 1e-12)
    q = d * (1.0 / RADIUS)
    w = jnp.maximum(1.0 - q * q * q, 0.0)
    A = jnp.where(sel, w, 0.0).astype(f32)
    B = jnp.where(sel & (D > 0), 1.0, 0.0).astype(f32)
    csel = jnp.sum(sel.astype(f32), axis=1, keepdims=True)

    pcl = pcl_ref[...]
    M = jnp.concatenate([pcl, pcl * pcl, jnp.ones((P, 1), f32)], axis=1)
    dot = functools.partial(jax.lax.dot_general,
                            dimension_numbers=(((1,), (0,)), ((), ())),
                            preferred_element_type=f32)
    GAB = dot(jnp.concatenate([A, B], axis=0), M)
    GA = GAB[:BR]
    GB = GAB[BR:]

    npad = jnp.maximum(float(K) - csel, 0.0)
    d0 = jnp.sqrt(xn + 1e-12)
    q0 = d0 * (1.0 / RADIUS)
    w0pad = jnp.maximum(1.0 - q0 * q0 * q0, 0.0)
    density = GA[:, 6:7] + npad * w0pad
    pos = GA[:, 0:3] / (density + 1e-12)
    num_nn = GB[:, 6:7]
    svec = GB[:, 0:3] - num_nn * X
    vmean = svec / (num_nn + 1e-12)
    svec2 = GB[:, 3:6] - 2.0 * X * GB[:, 0:3] + num_nn * (X * X)
    var = (svec2 - 2.0 * vmean * svec + num_nn * vmean * vmean) / (num_nn + 1e-12)

    ro = ro_ref[...]
    sd_raw = pos - ro
    sd = sd_raw / jnp.sqrt(jnp.sum(sd_raw * sd_raw, axis=1, keepdims=True))

    hdir = hdir_ref[...]
    bases = (X, density, pos, var, hdir, sd)
    scaled = jnp.concatenate(
        [b * (2.0 ** i) for b, (_, _, nf) in zip(bases, _GROUPS)
         for i in range(nf)], axis=1)
    feat = jnp.concatenate(
        list(bases) + [jnp.sin(scaled), jnp.cos(scaled),
                       nrm_ref[...], fv_ref[...], jnp.zeros((BR, 1), f32)],
        axis=1)

    dotd = functools.partial(jax.lax.dot_general,
                             dimension_numbers=(((1,), (0,)), ((), ())),
                             preferred_element_type=f32)
    h = jnp.maximum(dotd(feat, w0_ref[...]) + b0_ref[...], 0.0)
    h = jnp.maximum(dotd(h, w1_ref[...]) + b1_ref[...], 0.0)
    h = jnp.maximum(dotd(h, w2_ref[...]) + b2_ref[...], 0.0)
    h = jnp.maximum(dotd(h, w3_ref[...]) + b3_ref[...], 0.0)
    o = dotd(h, w4_ref[...]) + b4_ref[...]
    out_ref[...] = jax.nn.sigmoid(o)


def kernel(points, normals, view_dirs, feature_vectors, physical_particles,
           rays, ro, W0, b0, W1, b1, W2, b2, W3, b3, W4, b4):
    f32 = jnp.float32
    pts = points.reshape(N, 3)
    ptr = physical_particles.T
    hdir = jnp.repeat(rays[:, 3:], S, axis=0)

    W0t = jnp.concatenate([W0[:, _PERM].T, jnp.zeros((1, 512), f32)], axis=0)
    W4t = jnp.concatenate([W4.T, jnp.zeros((512, 125), f32)], axis=1)
    b4p = jnp.concatenate([b4, jnp.zeros((125,), f32)])

    grid = (N // BR,)
    row = lambda i: (i, 0)
    rep = lambda i: (0, 0)
    out = pl.pallas_call(
        _fused_kernel,
        grid=grid,
        in_specs=[
            pl.BlockSpec((BR, 3), row),
            pl.BlockSpec((3, P), rep),
            pl.BlockSpec((P, 3), rep),
            pl.BlockSpec((BR, 3), row),
            pl.BlockSpec((1, 3), rep),
            pl.BlockSpec((BR, 3), row),
            pl.BlockSpec((BR, FEAT), row),
            pl.BlockSpec((512, 512), rep),
            pl.BlockSpec((1, 512), rep),
            pl.BlockSpec((512, 512), rep),
            pl.BlockSpec((1, 512), rep),
            pl.BlockSpec((512, 512), rep),
            pl.BlockSpec((1, 512), rep),
            pl.BlockSpec((512, 512), rep),
            pl.BlockSpec((1, 512), rep),
            pl.BlockSpec((512, 128), rep),
            pl.BlockSpec((1, 128), rep),
        ],
        out_specs=pl.BlockSpec((BR, 128), row),
        out_shape=jax.ShapeDtypeStruct((N, 128), f32),
    )(pts, ptr, physical_particles, hdir, ro, normals, feature_vectors,
      W0t, b0.reshape(1, 512), W1.T, b1.reshape(1, 512), W2.T,
      b2.reshape(1, 512), W3.T, b3.reshape(1, 512), W4t, b4p.reshape(1, 128))
    return out[:, :3]


# speedup vs baseline: 3.0994x; 1.0728x over previous
import functools

import numpy as np
import jax
import jax.numpy as jnp
from jax.experimental import pallas as pl

R, S, P, K = 512, 16, 2048, 20
RADIUS = 4.0 * 0.05
FEAT = 256
N = R * S

_R2_F32 = np.float32(RADIUS) * np.float32(RADIUS)
_TMAX = int(np.asarray(_R2_F32, np.float32).view(np.int32)) - 1
_BITS = 30

BR = 512


def _emb_cols(offset, dim, nfreq):
    base = list(range(offset, offset + dim))
    sins, coss = [], []
    for i in range(nfreq):
        sins += list(range(offset + dim * (1 + 2 * i), offset + dim * (2 + 2 * i)))
        coss += list(range(offset + dim * (2 + 2 * i), offset + dim * (3 + 2 * i)))
    return base, sins, coss


_GROUPS = ((0, 3, 10), (63, 1, 4), (72, 3, 10), (135, 3, 10),
           (198, 3, 4), (225, 3, 4))


def _w0_permutation():
    perm = []
    for off, dg, _ in _GROUPS:
        perm += [off + d for d in range(dg)]
    for trig in (1, 2):
        for off, dg, nf in _GROUPS:
            for i in range(nf):
                perm += [off + dg * (trig + 2 * i) + d for d in range(dg)]
    perm += list(range(252, 255)) + list(range(255, 511))
    assert len(perm) == 511 and sorted(perm) == list(range(511))
    return np.asarray(perm, np.int32)


_PERM = _w0_permutation()


def _fused_kernel(pts_ref, ptr_ref, pcl_ref, hdir_ref, ro_ref, nrm_ref,
                  fv_ref, w0_ref, b0_ref, w1_ref, b1_ref, w2_ref, b2_ref,
                  w3_ref, b3_ref, w4_ref, b4_ref, out_ref):
    f32 = jnp.float32
    X = pts_ref[...]
    ptr = ptr_ref[...]
    xn = X[:, 0:1] * X[:, 0:1] + X[:, 1:2] * X[:, 1:2] + X[:, 2:3] * X[:, 2:3]
    pn = jnp.sum(ptr * ptr, axis=0, keepdims=True)
    ip_sel = jax.lax.dot_general(
        X.astype(jnp.bfloat16), ptr.astype(jnp.bfloat16),
        (((1,), (0,)), ((), ())), preferred_element_type=f32)
    d2s = jnp.maximum(xn + pn - 2.0 * ip_sel, 0.0)
    D = jax.lax.bitcast_convert_type(d2s, jnp.int32)
    ip = (X[:, 0:1] * ptr[0:1, :] + X[:, 1:2] * ptr[1:2, :]
          + X[:, 2:3] * ptr[2:3, :])
    d2 = jnp.maximum(xn + pn - 2.0 * ip, 0.0)

    one16 = jnp.ones((), jnp.bfloat16)
    zero16 = jnp.zeros((), jnp.bfloat16)
    Dq = (jnp.minimum(jax.lax.shift_right_logical(D, 14), 65535)
          - 32768).astype(jnp.int16)

    def cbody(_, carry):
        lo, hi = carry
        mid = jax.lax.shift_right_logical(lo + hi, 1)
        mid16 = (mid - 32768).astype(jnp.int16)
        mask16 = jnp.where(Dq <= mid16, one16, zero16)
        part = mask16[:, 0:128]
        for j in range(1, 16):
            part = part + mask16[:, 128 * j:128 * (j + 1)]
        cnt = jnp.sum(part.astype(f32), axis=1, keepdims=True)
        ge = cnt >= float(K)
        return jnp.where(ge, lo, mid + 1), jnp.where(ge, mid, hi)

    qmax = _TMAX >> 14
    lo0 = jnp.zeros((BR, 1), jnp.int32)
    hi0 = jnp.full((BR, 1), qmax, jnp.int32)
    mq, _ = jax.lax.fori_loop(0, 16, cbody, (lo0, hi0))
    mq = jnp.minimum(mq, qmax)

    elo0 = jax.lax.shift_left(mq, 14)
    ehi0 = jnp.minimum(elo0 + 0x3FFF, _TMAX)
    El = jnp.where(D < elo0, -32768,
                   jnp.where(D <= ehi0, (D & 0x3FFF) - 16384,
                             32767)).astype(jnp.int16)

    def body(_, carry):
        lo, hi = carry
        mid = jax.lax.shift_right_logical(lo + hi, 1)
        mid16 = (mid - 16384).astype(jnp.int16)
        mask16 = jnp.where(El <= mid16, one16, zero16)
        part = mask16[:, 0:128]
        for j in range(1, 16):
            part = part + mask16[:, 128 * j:128 * (j + 1)]
        cnt = jnp.sum(part.astype(f32), axis=1, keepdims=True)
        ge = cnt >= float(K)
        return jnp.where(ge, lo, mid + 1), jnp.where(ge, mid, hi)

    tl0 = jnp.zeros((BR, 1), jnp.int32)
    th0 = jnp.full((BR, 1), 0x3FFF, jnp.int32)
    tl, _ = jax.lax.fori_loop(0, 14, body, (tl0, th0))
    t = jnp.minimum(elo0 + tl, ehi0)

    sel = D <= t
    d = jnp.sqrt(d2 + 1e-12)
    q = d * (1.0 / RADIUS)
    w = jnp.maximum(1.0 - q * q * q, 0.0)
    A = jnp.where(sel, w, 0.0).astype(f32)
    B = jnp.where(sel & (D > 0), 1.0, 0.0).astype(f32)
    csel = jnp.sum(sel.astype(f32), axis=1, keepdims=True)

    pcl = pcl_ref[...]
    M = jnp.concatenate([pcl, pcl * pcl, jnp.ones((P, 1), f32)], axis=1)
    dot = functools.partial(jax.lax.dot_general,
                            dimension_numbers=(((1,), (0,)), ((), ())),
                            preferred_element_type=f32)
    GAB = dot(jnp.concatenate([A, B], axis=0), M)
    GA = GAB[:BR]
    GB = GAB[BR:]

    npad = jnp.maximum(float(K) - csel, 0.0)
    d0 = jnp.sqrt(xn + 1e-12)
    q0 = d0 * (1.0 / RADIUS)
    w0pad = jnp.maximum(1.0 - q0 * q0 * q0, 0.0)
    density = GA[:, 6:7] + npad * w0pad
    pos = GA[:, 0:3] / (density + 1e-12)
    num_nn = GB[:, 6:7]
    svec = GB[:, 0:3] - num_nn * X
    vmean = svec / (num_nn + 1e-12)
    svec2 = GB[:, 3:6] - 2.0 * X * GB[:, 0:3] + num_nn * (X * X)
    var = (svec2 - 2.0 * vmean * svec + num_nn * vmean * vmean) / (num_nn + 1e-12)

    ro = ro_ref[...]
    sd_raw = pos - ro
    sd = sd_raw / jnp.sqrt(jnp.sum(sd_raw * sd_raw, axis=1, keepdims=True))

    hdir = hdir_ref[...]
    bases = (X, density, pos, var, hdir, sd)
    scaled = jnp.concatenate(
        [b * (2.0 ** i) for b, (_, _, nf) in zip(bases, _GROUPS)
         for i in range(nf)], axis=1)
    feat = jnp.concatenate(
        list(bases) + [jnp.sin(scaled), jnp.cos(scaled),
                       nrm_ref[...], fv_ref[...], jnp.zeros((BR, 1), f32)],
        axis=1)

    dotd = functools.partial(jax.lax.dot_general,
                             dimension_numbers=(((1,), (0,)), ((), ())),
                             preferred_element_type=f32)
    h = jnp.maximum(dotd(feat, w0_ref[...]) + b0_ref[...], 0.0)
    h = jnp.maximum(dotd(h, w1_ref[...]) + b1_ref[...], 0.0)
    h = jnp.maximum(dotd(h, w2_ref[...]) + b2_ref[...], 0.0)
    h = jnp.maximum(dotd(h, w3_ref[...]) + b3_ref[...], 0.0)
    o = dotd(h, w4_ref[...]) + b4_ref[...]
    out_ref[...] = jax.nn.sigmoid(o)


def kernel(points, normals, view_dirs, feature_vectors, physical_particles,
           rays, ro, W0, b0, W1, b1, W2, b2, W3, b3, W4, b4):
    f32 = jnp.float32
    pts = points.reshape(N, 3)
    ptr = physical_particles.T
    hdir = jnp.repeat(rays[:, 3:], S, axis=0)

    W0t = jnp.concatenate([W0[:, _PERM].T, jnp.zeros((1, 512), f32)], axis=0)
    W4t = jnp.concatenate([W4.T, jnp.zeros((512, 125), f32)], axis=1)
    b4p = jnp.concatenate([b4, jnp.zeros((125,), f32)])

    grid = (N // BR,)
    row = lambda i: (i, 0)
    rep = lambda i: (0, 0)
    out = pl.pallas_call(
        _fused_kernel,
        grid=grid,
        in_specs=[
            pl.BlockSpec((BR, 3), row),
            pl.BlockSpec((3, P), rep),
            pl.BlockSpec((P, 3), rep),
            pl.BlockSpec((BR, 3), row),
            pl.BlockSpec((1, 3), rep),
            pl.BlockSpec((BR, 3), row),
            pl.BlockSpec((BR, FEAT), row),
            pl.BlockSpec((512, 512), rep),
            pl.BlockSpec((1, 512), rep),
            pl.BlockSpec((512, 512), rep),
            pl.BlockSpec((1, 512), rep),
            pl.BlockSpec((512, 512), rep),
            pl.BlockSpec((1, 512), rep),
            pl.BlockSpec((512, 512), rep),
            pl.BlockSpec((1, 512), rep),
            pl.BlockSpec((512, 128), rep),
            pl.BlockSpec((1, 128), rep),
        ],
        out_specs=pl.BlockSpec((BR, 128), row),
        out_shape=jax.ShapeDtypeStruct((N, 128), f32),
    )(pts, ptr, physical_particles, hdir, ro, normals, feature_vectors,
      W0t, b0.reshape(1, 512), W1.T, b1.reshape(1, 512), W2.T,
      b2.reshape(1, 512), W3.T, b3.reshape(1, 512), W4t, b4p.reshape(1, 128))
    return out[:, :3]


# speedup vs baseline: 3.1223x; 1.0074x over previous
import functools

import numpy as np
import jax
import jax.numpy as jnp
from jax.experimental import pallas as pl

R, S, P, K = 512, 16, 2048, 20
RADIUS = 4.0 * 0.05
FEAT = 256
N = R * S

_R2_F32 = np.float32(RADIUS) * np.float32(RADIUS)
_TMAX = int(np.asarray(_R2_F32, np.float32).view(np.int32)) - 1
_BITS = 30

BR = 1024


def _emb_cols(offset, dim, nfreq):
    base = list(range(offset, offset + dim))
    sins, coss = [], []
    for i in range(nfreq):
        sins += list(range(offset + dim * (1 + 2 * i), offset + dim * (2 + 2 * i)))
        coss += list(range(offset + dim * (2 + 2 * i), offset + dim * (3 + 2 * i)))
    return base, sins, coss


_GROUPS = ((0, 3, 10), (63, 1, 4), (72, 3, 10), (135, 3, 10),
           (198, 3, 4), (225, 3, 4))


def _w0_permutation():
    perm = []
    for off, dg, _ in _GROUPS:
        perm += [off + d for d in range(dg)]
    for trig in (1, 2):
        for off, dg, nf in _GROUPS:
            for i in range(nf):
                perm += [off + dg * (trig + 2 * i) + d for d in range(dg)]
    perm += list(range(252, 255)) + list(range(255, 511))
    assert len(perm) == 511 and sorted(perm) == list(range(511))
    return np.asarray(perm, np.int32)


_PERM = _w0_permutation()


def _fused_kernel(pts_ref, ptr_ref, pcl_ref, hdir_ref, ro_ref, nrm_ref,
                  fv_ref, w0_ref, b0_ref, w1_ref, b1_ref, w2_ref, b2_ref,
                  w3_ref, b3_ref, w4_ref, b4_ref, out_ref):
    f32 = jnp.float32
    X = pts_ref[...]
    ptr = ptr_ref[...]
    xn = X[:, 0:1] * X[:, 0:1] + X[:, 1:2] * X[:, 1:2] + X[:, 2:3] * X[:, 2:3]
    pn = jnp.sum(ptr * ptr, axis=0, keepdims=True)
    ip_sel = jax.lax.dot_general(
        X.astype(jnp.bfloat16), ptr.astype(jnp.bfloat16),
        (((1,), (0,)), ((), ())), preferred_element_type=f32)
    d2s = jnp.maximum(xn + pn - 2.0 * ip_sel, 0.0)
    D = jax.lax.bitcast_convert_type(d2s, jnp.int32)
    ip = (X[:, 0:1] * ptr[0:1, :] + X[:, 1:2] * ptr[1:2, :]
          + X[:, 2:3] * ptr[2:3, :])
    d2 = jnp.maximum(xn + pn - 2.0 * ip, 0.0)

    one16 = jnp.ones((), jnp.bfloat16)
    zero16 = jnp.zeros((), jnp.bfloat16)
    Dq = (jnp.minimum(jax.lax.shift_right_logical(D, 14), 65535)
          - 32768).astype(jnp.int16)

    def cbody(_, carry):
        lo, hi = carry
        mid = jax.lax.shift_right_logical(lo + hi, 1)
        mid16 = (mid - 32768).astype(jnp.int16)
        mask16 = jnp.where(Dq <= mid16, one16, zero16)
        part = mask16[:, 0:128]
        for j in range(1, 16):
            part = part + mask16[:, 128 * j:128 * (j + 1)]
        cnt = jnp.sum(part.astype(f32), axis=1, keepdims=True)
        ge = cnt >= float(K)
        return jnp.where(ge, lo, mid + 1), jnp.where(ge, mid, hi)

    qmax = _TMAX >> 14
    lo0 = jnp.zeros((BR, 1), jnp.int32)
    hi0 = jnp.full((BR, 1), qmax, jnp.int32)
    mq, _ = jax.lax.fori_loop(0, 16, cbody, (lo0, hi0))
    mq = jnp.minimum(mq, qmax)

    elo0 = jax.lax.shift_left(mq, 14)
    ehi0 = jnp.minimum(elo0 + 0x3FFF, _TMAX)
    El = jnp.where(D < elo0, -32768,
                   jnp.where(D <= ehi0, (D & 0x3FFF) - 16384,
                             32767)).astype(jnp.int16)

    def body(_, carry):
        lo, hi = carry
        mid = jax.lax.shift_right_logical(lo + hi, 1)
        mid16 = (mid - 16384).astype(jnp.int16)
        mask16 = jnp.where(El <= mid16, one16, zero16)
        part = mask16[:, 0:128]
        for j in range(1, 16):
            part = part + mask16[:, 128 * j:128 * (j + 1)]
        cnt = jnp.sum(part.astype(f32), axis=1, keepdims=True)
        ge = cnt >= float(K)
        return jnp.where(ge, lo, mid + 1), jnp.where(ge, mid, hi)

    tl0 = jnp.zeros((BR, 1), jnp.int32)
    th0 = jnp.full((BR, 1), 0x3FFF, jnp.int32)
    tl, _ = jax.lax.fori_loop(0, 14, body, (tl0, th0))
    t = jnp.minimum(elo0 + tl, ehi0)

    sel = D <= t
    d = jnp.sqrt(d2 + 1e-12)
    q = d * (1.0 / RADIUS)
    w = jnp.maximum(1.0 - q * q * q, 0.0)
    A = jnp.where(sel, w, 0.0).astype(f32)
    B = jnp.where(sel & (D > 0), 1.0, 0.0).astype(f32)
    csel = jnp.sum(sel.astype(f32), axis=1, keepdims=True)

    pcl = pcl_ref[...]
    M = jnp.concatenate([pcl, pcl * pcl, jnp.ones((P, 1), f32)], axis=1)
    dot = functools.partial(jax.lax.dot_general,
                            dimension_numbers=(((1,), (0,)), ((), ())),
                            preferred_element_type=f32)
    GAB = dot(jnp.concatenate([A, B], axis=0), M)
    GA = GAB[:BR]
    GB = GAB[BR:]

    npad = jnp.maximum(float(K) - csel, 0.0)
    d0 = jnp.sqrt(xn + 1e-12)
    q0 = d0 * (1.0 / RADIUS)
    w0pad = jnp.maximum(1.0 - q0 * q0 * q0, 0.0)
    density = GA[:, 6:7] + npad * w0pad
    pos = GA[:, 0:3] / (density + 1e-12)
    num_nn = GB[:, 6:7]
    svec = GB[:, 0:3] - num_nn * X
    vmean = svec / (num_nn + 1e-12)
    svec2 = GB[:, 3:6] - 2.0 * X * GB[:, 0:3] + num_nn * (X * X)
    var = (svec2 - 2.0 * vmean * svec + num_nn * vmean * vmean) / (num_nn + 1e-12)

    ro = ro_ref[...]
    sd_raw = pos - ro
    sd = sd_raw / jnp.sqrt(jnp.sum(sd_raw * sd_raw, axis=1, keepdims=True))

    hdir = hdir_ref[...]
    bases = (X, density, pos, var, hdir, sd)
    scaled = jnp.concatenate(
        [b * (2.0 ** i) for b, (_, _, nf) in zip(bases, _GROUPS)
         for i in range(nf)], axis=1)
    feat = jnp.concatenate(
        list(bases) + [jnp.sin(scaled), jnp.cos(scaled),
                       nrm_ref[...], fv_ref[...], jnp.zeros((BR, 1), f32)],
        axis=1)

    dotd = functools.partial(jax.lax.dot_general,
                             dimension_numbers=(((1,), (0,)), ((), ())),
                             preferred_element_type=f32)
    h = jnp.maximum(dotd(feat, w0_ref[...]) + b0_ref[...], 0.0)
    h = jnp.maximum(dotd(h, w1_ref[...]) + b1_ref[...], 0.0)
    h = jnp.maximum(dotd(h, w2_ref[...]) + b2_ref[...], 0.0)
    h = jnp.maximum(dotd(h, w3_ref[...]) + b3_ref[...], 0.0)
    o = dotd(h, w4_ref[...]) + b4_ref[...]
    out_ref[...] = jax.nn.sigmoid(o)


def kernel(points, normals, view_dirs, feature_vectors, physical_particles,
           rays, ro, W0, b0, W1, b1, W2, b2, W3, b3, W4, b4):
    f32 = jnp.float32
    pts = points.reshape(N, 3)
    ptr = physical_particles.T
    hdir = jnp.repeat(rays[:, 3:], S, axis=0)

    W0t = jnp.concatenate([W0[:, _PERM].T, jnp.zeros((1, 512), f32)], axis=0)
    W4t = jnp.concatenate([W4.T, jnp.zeros((512, 125), f32)], axis=1)
    b4p = jnp.concatenate([b4, jnp.zeros((125,), f32)])

    grid = (N // BR,)
    row = lambda i: (i, 0)
    rep = lambda i: (0, 0)
    out = pl.pallas_call(
        _fused_kernel,
        grid=grid,
        in_specs=[
            pl.BlockSpec((BR, 3), row),
            pl.BlockSpec((3, P), rep),
            pl.BlockSpec((P, 3), rep),
            pl.BlockSpec((BR, 3), row),
            pl.BlockSpec((1, 3), rep),
            pl.BlockSpec((BR, 3), row),
            pl.BlockSpec((BR, FEAT), row),
            pl.BlockSpec((512, 512), rep),
            pl.BlockSpec((1, 512), rep),
            pl.BlockSpec((512, 512), rep),
            pl.BlockSpec((1, 512), rep),
            pl.BlockSpec((512, 512), rep),
            pl.BlockSpec((1, 512), rep),
            pl.BlockSpec((512, 512), rep),
            pl.BlockSpec((1, 512), rep),
            pl.BlockSpec((512, 128), rep),
            pl.BlockSpec((1, 128), rep),
        ],
        out_specs=pl.BlockSpec((BR, 128), row),
        out_shape=jax.ShapeDtypeStruct((N, 128), f32),
    )(pts, ptr, physical_particles, hdir, ro, normals, feature_vectors,
      W0t, b0.reshape(1, 512), W1.T, b1.reshape(1, 512), W2.T,
      b2.reshape(1, 512), W3.T, b3.reshape(1, 512), W4t, b4p.reshape(1, 128))
    return out[:, :3]
